# Initial kernel scaffold; baseline (speedup 1.0000x reference)
import jax
import jax.numpy as jnp
from jax.experimental import pallas as pl


def kernel(scalar_feat, vec_feat, r_feat, rel_x, x, ligand_emb, edge_feat, edge_index, ligand_shape, invar_ligand_shape, params):
    raise NotImplementedError("write your pallas kernel here")


# speedup vs baseline: 14.1867x; 14.1867x over previous
import functools

import jax
import jax.numpy as jnp
import numpy as np
from jax import lax
from jax.experimental import pallas as pl
from jax.experimental.pallas import tpu as pltpu
from jax.experimental.pallas import tpu_sc as plsc

EPS = 1e-8
SLOPE = 0.2


def _ln(x, g, b):
    m = jnp.mean(x, axis=-1, keepdims=True)
    v = jnp.mean((x - m) ** 2, axis=-1, keepdims=True)
    return (x - m) / jnp.sqrt(v + 1e-5) * g + b


def _full_spec(shape):
    nd = len(shape)
    return pl.BlockSpec(shape, lambda i, _nd=nd: (0,) * _nd)


def _row_spec(bn, d):
    return pl.BlockSpec((bn, d), lambda i: (i, 0))


def _node_pre_body(sf_r, vfx_r, vfy_r, vfz_r, lsx_r, lsy_r, lsz_r, iv_r,
                   As_r, An_r, Avn_r, Aiv_r, b1_r, W2_r, b2_r,
                   S1_r, Sb1_r, Sg_r, Sbe_r, S2_r, Sb2_r,
                   V1_r, V1b_r, V2_r, V2b_r,
                   WT_r, UT_r,
                   Q1_r, Qb1_r, Qg_r, Qbe_r, Q2_r, Qb2_r,
                   st_ref, q_ref):
    sf = sf_r[...]
    vx, vy, vz = vfx_r[...], vfy_r[...], vfz_r[...]
    lx, ly, lz = lsx_r[...], lsy_r[...], lsz_r[...]
    vn_vf = jnp.sqrt(vx * vx + vy * vy + vz * vz + EPS)
    acc = sf @ As_r[...] + vn_vf @ Avn_r[...] + iv_r[...] @ Aiv_r[...] + b1_r[...]
    An = An_r[...]
    for m in range(16):
        nm = lx[:, m:m + 1] * vx + ly[:, m:m + 1] * vy + lz[:, m:m + 1] * vz
        acc = acc + nm @ An[16 * m:16 * (m + 1), :]
    o = jnp.maximum(acc, 0.0) @ W2_r[...] + b2_r[...]
    S1 = S1_r[...]
    h = sf @ S1[0:128] + o @ S1[128:256] + Sb1_r[...]
    h = _ln(h, Sg_r[...], Sbe_r[...])
    se = jnp.maximum(h, 0.0) @ S2_r[...] + Sb2_r[...]
    t = jnp.maximum(o @ V1_r[...] + V1b_r[...], 0.0)
    vo2 = t @ V2_r[...] + V2b_r[...]
    vmx = jnp.zeros_like(vx)
    vmy = jnp.zeros_like(vy)
    vmz = jnp.zeros_like(vz)
    for s in range(16):
        blk = vo2[:, 16 * s:16 * (s + 1)]
        vmx = vmx + blk * lx[:, s:s + 1]
        vmy = vmy + blk * ly[:, s:s + 1]
        vmz = vmz + blk * lz[:, s:s + 1]
    WT = WT_r[...]
    UT = UT_r[...]
    pmx = vx @ WT[0:16] + vmx @ WT[16:32]
    pmy = vy @ WT[0:16] + vmy @ WT[16:32]
    pmz = vz @ WT[0:16] + vmz @ WT[16:32]
    ddx = vx @ UT[0:16] + vmx @ UT[16:32]
    ddy = vy @ UT[0:16] + vmy @ UT[16:32]
    ddz = vz @ UT[0:16] + vmz @ UT[16:32]
    dot = pmx * ddx + pmy * ddy + pmz * ddz
    d2 = ddx * ddx + ddy * ddy + ddz * ddz
    f = dot / (d2 + EPS)
    ok = dot >= 0.0
    vex = SLOPE * pmx + (1.0 - SLOPE) * jnp.where(ok, pmx, pmx - f * ddx)
    vey = SLOPE * pmy + (1.0 - SLOPE) * jnp.where(ok, pmy, pmy - f * ddy)
    vez = SLOPE * pmz + (1.0 - SLOPE) * jnp.where(ok, pmz, pmz - f * ddz)
    vn_ve = jnp.sqrt(vex * vex + vey * vey + vez * vez + EPS)
    Q1 = Q1_r[...]
    hq = se @ Q1[0:128] + vn_ve @ Q1[128:144] + Qb1_r[...]
    hq = _ln(hq, Qg_r[...], Qbe_r[...])
    qv = jnp.maximum(hq, 0.0) @ Q2_r[...] + Qb2_r[...]
    st_ref[:, 0:128] = se
    st_ref[:, 128:144] = vex
    st_ref[:, 144:160] = vey
    st_ref[:, 160:176] = vez
    q_ref[...] = qv


def _node_pre(sf, vfx, vfy, vfz, lsx, lsy, lsz, iv, wts):
    n = sf.shape[0]
    bn = 1000 if n % 1000 == 0 else n
    grid = (n // bn,)
    node_ins = [sf, vfx, vfy, vfz, lsx, lsy, lsz, iv]
    node_specs = [_row_spec(bn, a.shape[1]) for a in node_ins]
    w_specs = [_full_spec(w.shape) for w in wts]
    return pl.pallas_call(
        _node_pre_body,
        grid=grid,
        in_specs=node_specs + w_specs,
        out_specs=[_row_spec(bn, 256), _row_spec(bn, 128)],
        out_shape=[jax.ShapeDtypeStruct((n, 256), jnp.float32),
                   jax.ShapeDtypeStruct((n, 128), jnp.float32)],
    )(*node_ins, *wts)


def _edge_gather(src_table, qtab, src, dst):
    e = src.shape[0]
    nw = 32
    per = e // nw
    c = 80
    nch = per // c
    mesh = plsc.VectorSubcoreMesh(core_axis_name="c", subcore_axis_name="s")

    @functools.partial(
        pl.kernel,
        out_type=(jax.ShapeDtypeStruct((e, 256), jnp.float32),
                  jax.ShapeDtypeStruct((e, 128), jnp.float32)),
        mesh=mesh,
        scratch_types=[pltpu.VMEM((c,), jnp.int32), pltpu.VMEM((c,), jnp.int32),
                       pltpu.VMEM((c, 256), jnp.float32),
                       pltpu.VMEM((c, 128), jnp.float32),
                       pltpu.SemaphoreType.DMA],
    )
    def k(st_hbm, qt_hbm, src_hbm, dst_hbm, sg_hbm, qd_hbm,
          si_v, di_v, r1_v, r2_v, sem):
        wid = lax.axis_index("c") * 16 + lax.axis_index("s")
        base = wid * per

        def body(i, carry):
            off = base + i * c
            pltpu.sync_copy(src_hbm.at[pl.ds(off, c)], si_v)
            pltpu.sync_copy(dst_hbm.at[pl.ds(off, c)], di_v)
            d1 = pltpu.async_copy(st_hbm.at[si_v], r1_v, sem)
            d2 = pltpu.async_copy(qt_hbm.at[di_v], r2_v, sem)
            d1.wait()
            d2.wait()
            pltpu.sync_copy(r1_v, sg_hbm.at[pl.ds(off, c)])
            pltpu.sync_copy(r2_v, qd_hbm.at[pl.ds(off, c)])
            return carry

        lax.fori_loop(0, nch, body, 0)

    return k(src_table, qtab, src, dst)


def _edge_body(sg_r, qd_r, rf_r, ef_r, rx_r,
               Wh1a_r, Wh1b_r, Ws1a_r, Ws1b_r, Ws1c_r, Ws1d_r, bs1_r,
               Wv1_r, Wg1_r, bg1_r,
               Wh2_r, Ws2a_r, Ws2b_r, bs2_r, Wv2_r, Wg2_r, bg2_r,
               Wh3_r, Ws3a_r, Ws3b_r, bs3_r, Wv3_r, Wg3_r, bg3_r,
               K1a_r, K1b_r, Kb1_r, Kg_r, Kbe_r, K2_r, Kb2_r,
               fm_ref, fv_ref, lg_ref, lmax_ref):
    sg = sg_r[...]
    sc = sg[:, 0:128]
    vsx = sg[:, 128:144]
    vsy = sg[:, 144:160]
    vsz = sg[:, 160:176]
    rx = rx_r[...]
    Wh1b = Wh1b_r[...]
    vhx = vsx @ Wh1a_r[...] + rx[:, 0:1] * Wh1b
    vhy = vsy @ Wh1a_r[...] + rx[:, 1:2] * Wh1b
    vhz = vsz @ Wh1a_r[...] + rx[:, 2:3] * Wh1b
    vn = jnp.sqrt(vhx * vhx + vhy * vhy + vhz * vhz + EPS)
    s = jnp.maximum(sc @ Ws1a_r[...] + rf_r[...] @ Ws1b_r[...]
                    + ef_r[...] @ Ws1c_r[...] + vn @ Ws1d_r[...] + bs1_r[...], 0.0)
    gate = jax.nn.sigmoid(s @ Wg1_r[...] + bg1_r[...])
    vx = (vhx @ Wv1_r[...]) * gate
    vy = (vhy @ Wv1_r[...]) * gate
    vz = (vhz @ Wv1_r[...]) * gate
    vhx = vx @ Wh2_r[...]
    vhy = vy @ Wh2_r[...]
    vhz = vz @ Wh2_r[...]
    vn = jnp.sqrt(vhx * vhx + vhy * vhy + vhz * vhz + EPS)
    s = jnp.maximum(s @ Ws2a_r[...] + vn @ Ws2b_r[...] + bs2_r[...], 0.0)
    gate = jax.nn.sigmoid(s @ Wg2_r[...] + bg2_r[...])
    vx = (vhx @ Wv2_r[...]) * gate
    vy = (vhy @ Wv2_r[...]) * gate
    vz = (vhz @ Wv2_r[...]) * gate
    vhx = vx @ Wh3_r[...]
    vhy = vy @ Wh3_r[...]
    vhz = vz @ Wh3_r[...]
    vn = jnp.sqrt(vhx * vhx + vhy * vhy + vhz * vhz + EPS)
    s = jnp.maximum(s @ Ws3a_r[...] + vn @ Ws3b_r[...] + bs3_r[...], 0.0)
    gate = jax.nn.sigmoid(s @ Wg3_r[...] + bg3_r[...])
    vx = (vhx @ Wv3_r[...]) * gate
    vy = (vhy @ Wv3_r[...]) * gate
    vz = (vhz @ Wv3_r[...]) * gate
    vn = jnp.sqrt(vx * vx + vy * vy + vz * vz + EPS)
    h = s @ K1a_r[...] + vn @ K1b_r[...] + Kb1_r[...]
    h = _ln(h, Kg_r[...], Kbe_r[...])
    kk = jnp.maximum(h, 0.0) @ K2_r[...] + Kb2_r[...]
    lg = jnp.sum(qd_r[...] * kk, axis=1, keepdims=True) * float(1.0 / np.sqrt(128.0))
    fm_ref[...] = s
    fv_ref[:, 0:16] = vx
    fv_ref[:, 16:32] = vy
    fv_ref[:, 32:48] = vz
    one0 = (lax.broadcasted_iota(jnp.int32, (vx.shape[0], 16), 1) == 0)
    fv_ref[:, 48:64] = one0.astype(jnp.float32)
    fv_ref[:, 64:128] = jnp.zeros((vx.shape[0], 64), jnp.float32)
    lg_ref[...] = lg
    bmax = jnp.max(lg, axis=0, keepdims=True)
    lmax_ref[...] = jnp.where(pl.program_id(0) == 0, bmax,
                              jnp.maximum(lmax_ref[...], bmax))


def _edge_compute(sg, qd, rf, ef, rx, wts):
    e = sg.shape[0]
    be = 2000 if e % 2000 == 0 else e
    nblk = e // be
    edge_ins = [sg, qd, rf, ef, rx]
    edge_specs = [_row_spec(be, a.shape[1]) for a in edge_ins]
    w_specs = [_full_spec(w.shape) for w in wts]
    return pl.pallas_call(
        _edge_body,
        grid=(nblk,),
        in_specs=edge_specs + w_specs,
        out_specs=[_row_spec(be, 128), _row_spec(be, 128), _row_spec(be, 1),
                   pl.BlockSpec((1, 1), lambda i: (0, 0))],
        out_shape=[jax.ShapeDtypeStruct((e, 128), jnp.float32),
                   jax.ShapeDtypeStruct((e, 128), jnp.float32),
                   jax.ShapeDtypeStruct((e, 1), jnp.float32),
                   jax.ShapeDtypeStruct((1, 1), jnp.float32)],
    )(*edge_ins, *wts)


def _edge_scale_body(fm_r, fv_r, lg_r, gm_r, fms_ref, fvs_ref):
    ex = jnp.exp(lg_r[...] - gm_r[...])
    fms_ref[...] = fm_r[...] * ex
    fvs_ref[...] = fv_r[...] * ex


def _edge_scale(fm, fv, lg, gmax11):
    e = fm.shape[0]
    be = 8000 if e % 8000 == 0 else e
    return pl.pallas_call(
        _edge_scale_body,
        grid=(e // be,),
        in_specs=[_row_spec(be, 128), _row_spec(be, 128), _row_spec(be, 1),
                  pl.BlockSpec((1, 1), lambda i: (0, 0))],
        out_specs=[_row_spec(be, 128), _row_spec(be, 128)],
        out_shape=[jax.ShapeDtypeStruct((e, 128), jnp.float32),
                   jax.ShapeDtypeStruct((e, 128), jnp.float32)],
    )(fm, fv, lg, gmax11)


def _edge_aggregate(fms, fvs, dst, zrows):
    e = fms.shape[0]
    npad = zrows.shape[0]
    nw = 32
    per = e // nw
    c = 80
    nch = per // c
    rpt = npad // 16
    mesh = plsc.VectorSubcoreMesh(core_axis_name="c", subcore_axis_name="s")

    @functools.partial(
        pl.kernel,
        out_type=jax.ShapeDtypeStruct((2, 2, npad, 128), jnp.float32),
        mesh=mesh,
        scratch_types=[pltpu.VMEM((c,), jnp.int32),
                       pltpu.VMEM((c, 128), jnp.float32),
                       pltpu.VMEM_SHARED((npad, 128), jnp.float32),
                       pltpu.SemaphoreType.DMA],
    )
    def k(fms_hbm, fvs_hbm, dst_hbm, z_hbm, out_hbm, idxb, featb, shared, sem):
        cid = lax.axis_index("c")
        sid = lax.axis_index("s")
        base = (cid * 16 + sid) * per

        for rnd, src_hbm in enumerate((fms_hbm, fvs_hbm)):
            pltpu.sync_copy(z_hbm.at[pl.ds(sid * rpt, rpt)],
                            shared.at[pl.ds(sid * rpt, rpt)])
            plsc.subcore_barrier()

            def chunk(i, carry, _src=src_hbm):
                off = base + i * c
                pltpu.sync_copy(dst_hbm.at[pl.ds(off, c)], idxb)
                pltpu.sync_copy(_src.at[pl.ds(off, c)], featb)
                pltpu.sync_copy(featb, shared.at[idxb], add=True)
                return carry

            lax.fori_loop(0, nch, chunk, 0)
            plsc.subcore_barrier()
            pltpu.sync_copy(shared.at[pl.ds(sid * rpt, rpt)],
                            out_hbm.at[cid, rnd, pl.ds(sid * rpt, rpt)])
            plsc.subcore_barrier()

    return k(fms, fvs, dst, zrows)


def _node_post_body(p00_r, p10_r, p01_r, p11_r, st_r, lig_r, sf_r, vfx_r, vfy_r, vfz_r, xx_r,
                    NA_r, NB_r, NC_r, W1_r, W2_r, W3_r, W4_r, nb1_r,
                    Nv1_r, Ng1_r, ngb1_r,
                    Mh2_r, M2a_r, M2b_r, mb2_r, Mv2_r, Mg2_r, mgb2_r,
                    Mh3_r, M3a_r, M3b_r, mb3_r, Mv3_r, Mg3_r, mgb3_r,
                    G_r, B_r,
                    s_ref, vxo_ref, vyo_ref, vzo_ref):
    aggs = p00_r[...] + p10_r[...]
    aggv = p01_r[...] + p11_r[...]
    inv = 1.0 / (aggv[:, 48:49] + EPS)
    sagg = aggs * inv
    vax = aggv[:, 0:16] * inv
    vay = aggv[:, 16:32] * inv
    vaz = aggv[:, 32:48] * inv
    st = st_r[...]
    se = st[:, 0:128]
    vex = st[:, 128:144]
    vey = st[:, 144:160]
    vez = st[:, 160:176]
    xx = xx_r[...]
    NA = NA_r[...]
    vhx = xx[:, 0:1] * NA + vex @ NB_r[...] + vax @ NC_r[...]
    vhy = xx[:, 1:2] * NA + vey @ NB_r[...] + vay @ NC_r[...]
    vhz = xx[:, 2:3] * NA + vez @ NB_r[...] + vaz @ NC_r[...]
    vn = jnp.sqrt(vhx * vhx + vhy * vhy + vhz * vhz + EPS)
    s = jnp.maximum(lig_r[...] @ W1_r[...] + se @ W2_r[...]
                    + sagg @ W3_r[...] + vn @ W4_r[...] + nb1_r[...], 0.0)
    gate = jax.nn.sigmoid(s @ Ng1_r[...] + ngb1_r[...])
    vx = (vhx @ Nv1_r[...]) * gate
    vy = (vhy @ Nv1_r[...]) * gate
    vz = (vhz @ Nv1_r[...]) * gate
    vhx = vx @ Mh2_r[...]
    vhy = vy @ Mh2_r[...]
    vhz = vz @ Mh2_r[...]
    vn = jnp.sqrt(vhx * vhx + vhy * vhy + vhz * vhz + EPS)
    s = jnp.maximum(s @ M2a_r[...] + vn @ M2b_r[...] + mb2_r[...], 0.0)
    gate = jax.nn.sigmoid(s @ Mg2_r[...] + mgb2_r[...])
    vx = (vhx @ Mv2_r[...]) * gate
    vy = (vhy @ Mv2_r[...]) * gate
    vz = (vhz @ Mv2_r[...]) * gate
    vhx = vx @ Mh3_r[...]
    vhy = vy @ Mh3_r[...]
    vhz = vz @ Mh3_r[...]
    vn = jnp.sqrt(vhx * vhx + vhy * vhy + vhz * vhz + EPS)
    s = jnp.maximum(s @ M3a_r[...] + vn @ M3b_r[...] + mb3_r[...], 0.0)
    gate = jax.nn.sigmoid(s @ Mg3_r[...] + mgb3_r[...])
    vx = (vhx @ Mv3_r[...]) * gate
    vy = (vhy @ Mv3_r[...]) * gate
    vz = (vhz @ Mv3_r[...]) * gate
    s = sf_r[...] + s
    s = _ln(s, G_r[...], B_r[...])
    vx = vfx_r[...] + vx
    vy = vfy_r[...] + vy
    vz = vfz_r[...] + vz
    vn2 = jnp.sum(vx * vx + vy * vy + vz * vz, axis=1, keepdims=True) * (1.0 / 16.0)
    r = 1.0 / jnp.sqrt(vn2 + EPS)
    s_ref[...] = s
    vxo_ref[...] = vx * r
    vyo_ref[...] = vy * r
    vzo_ref[...] = vz * r


def _node_post(p00, p10, p01, p11, st, lig, sf, vfx, vfy, vfz, xx, wts):
    n = sf.shape[0]
    bn = 1000 if n % 1000 == 0 else n
    node_ins = [p00, p10, p01, p11, st, lig, sf, vfx, vfy, vfz, xx]
    node_specs = [_row_spec(bn, a.shape[1]) for a in node_ins]
    w_specs = [_full_spec(w.shape) for w in wts]
    return pl.pallas_call(
        _node_post_body,
        grid=(n // bn,),
        in_specs=node_specs + w_specs,
        out_specs=[_row_spec(bn, 128), _row_spec(bn, 16),
                   _row_spec(bn, 16), _row_spec(bn, 16)],
        out_shape=[jax.ShapeDtypeStruct((n, 128), jnp.float32),
                   jax.ShapeDtypeStruct((n, 16), jnp.float32),
                   jax.ShapeDtypeStruct((n, 16), jnp.float32),
                   jax.ShapeDtypeStruct((n, 16), jnp.float32)],
    )(*node_ins, *wts)


def _prep_pre_weights(p):
    se1w = p['se1']['w']
    wts = [se1w[0:128], se1w[128:384], se1w[384:400], se1w[400:416],
           p['se1']['b'][None], p['se2']['w'], p['se2']['b'][None]]
    ss = p['shape_scalar']
    wts += [ss['l1']['w'], ss['l1']['b'][None], ss['g'][None], ss['be'][None],
            ss['l2']['w'], ss['l2']['b'][None]]
    perm = np.arange(256).reshape(16, 16).T.reshape(-1)
    wts += [p['sv1']['w'], p['sv1']['b'][None],
            p['sv2']['w'][:, perm], p['sv2']['b'][perm][None]]
    wts += [p['vec_emb']['W'].T, p['vec_emb']['U'].T]
    hq = p['hq']
    wts += [hq['l1']['w'], hq['l1']['b'][None], hq['g'][None], hq['be'][None],
            hq['l2']['w'], hq['l2']['b'][None]]
    return wts


def _prep_edge_weights(p):
    g1, g2, g3 = p['mess']
    wh1t = g1['wh'].T
    ws1 = g1['ws']['w']
    wts = [wh1t[0:16], wh1t[16:17],
           ws1[0:128], ws1[128:208], ws1[208:212], ws1[212:228],
           g1['ws']['b'][None], g1['wv'].T, g1['wg']['w'], g1['wg']['b'][None]]
    for g in (g2, g3):
        ws = g['ws']['w']
        wts += [g['wh'].T, ws[0:128], ws[128:144], g['ws']['b'][None],
                g['wv'].T, g['wg']['w'], g['wg']['b'][None]]
    hk = p['hk']
    k1 = hk['l1']['w']
    wts += [k1[0:128], k1[128:144], hk['l1']['b'][None], hk['g'][None],
            hk['be'][None], hk['l2']['w'], hk['l2']['b'][None]]
    return wts


def _prep_node_weights(p):
    n1, n2, n3 = p['node']
    wh1t = n1['wh'].T
    ws1 = n1['ws']['w']
    wts = [wh1t[0:1], wh1t[1:17], wh1t[17:33],
           ws1[0:128], ws1[128:256], ws1[256:384], ws1[384:512],
           n1['ws']['b'][None], n1['wv'].T, n1['wg']['w'], n1['wg']['b'][None]]
    for g in (n2, n3):
        ws = g['ws']['w']
        wts += [g['wh'].T, ws[0:128], ws[128:256], g['ws']['b'][None],
                g['wv'].T, g['wg']['w'], g['wg']['b'][None]]
    wts += [p['nn_g'][None], p['nn_b'][None]]
    return wts


def kernel(scalar_feat, vec_feat, r_feat, rel_x, x, ligand_emb, edge_feat,
           edge_index, ligand_shape, invar_ligand_shape, params):
    n = scalar_feat.shape[0]
    e = r_feat.shape[0]
    vft = jnp.transpose(vec_feat, (2, 0, 1))
    lst = jnp.transpose(ligand_shape, (2, 0, 1))
    src = edge_index[0].astype(jnp.int32)
    dst = edge_index[1].astype(jnp.int32)

    st, q = _node_pre(scalar_feat, vft[0], vft[1], vft[2],
                      lst[0], lst[1], lst[2], invar_ligand_shape,
                      _prep_pre_weights(params))
    sg, qd = _edge_gather(st, q, src, dst)
    fm, fv, lg, lmax = _edge_compute(sg, qd, r_feat, edge_feat, rel_x,
                                     _prep_edge_weights(params))
    fms, fvs = _edge_scale(fm, fv, lg, lmax)
    npad = 10240 if n <= 10240 else ((n + 127) // 128) * 128
    zrows = jnp.zeros((npad, 128), jnp.float32)
    partials = _edge_aggregate(fms, fvs, dst, zrows)

    s, vx, vy, vz = _node_post(partials[0, 0, :n], partials[1, 0, :n],
                               partials[0, 1, :n], partials[1, 1, :n],
                               st, ligand_emb,
                               scalar_feat, vft[0], vft[1], vft[2],
                               x.reshape(n, 3), _prep_node_weights(params))
    v = jnp.stack([vx, vy, vz], axis=-1)
    return (s, v)


# speedup vs baseline: 15.4429x; 1.0885x over previous
import functools

import jax
import jax.numpy as jnp
import numpy as np
from jax import lax
from jax.experimental import pallas as pl
from jax.experimental.pallas import tpu as pltpu
from jax.experimental.pallas import tpu_sc as plsc

EPS = 1e-8
SLOPE = 0.2


def _ln(x, g, b):
    m = jnp.mean(x, axis=-1, keepdims=True)
    v = jnp.mean((x - m) ** 2, axis=-1, keepdims=True)
    return (x - m) / jnp.sqrt(v + 1e-5) * g + b


def _full_spec(shape):
    nd = len(shape)
    return pl.BlockSpec(shape, lambda i, _nd=nd: (0,) * _nd)


def _row_spec(bn, d):
    return pl.BlockSpec((bn, d), lambda i: (i, 0))


def _node_pre_body(sf_r, vfx_r, vfy_r, vfz_r, lsx_r, lsy_r, lsz_r, iv_r,
                   As_r, An_r, Avn_r, Aiv_r, b1_r, W2_r, b2_r,
                   S1_r, Sb1_r, Sg_r, Sbe_r, S2_r, Sb2_r,
                   V1_r, V1b_r, V2_r, V2b_r,
                   WT_r, UT_r,
                   Q1_r, Qb1_r, Qg_r, Qbe_r, Q2_r, Qb2_r,
                   st_ref, q_ref):
    sf = sf_r[...]
    vx, vy, vz = vfx_r[...], vfy_r[...], vfz_r[...]
    lx, ly, lz = lsx_r[...], lsy_r[...], lsz_r[...]
    vn_vf = jnp.sqrt(vx * vx + vy * vy + vz * vz + EPS)
    acc = sf @ As_r[...] + vn_vf @ Avn_r[...] + iv_r[...] @ Aiv_r[...] + b1_r[...]
    An = An_r[...]
    for m in range(16):
        nm = lx[:, m:m + 1] * vx + ly[:, m:m + 1] * vy + lz[:, m:m + 1] * vz
        acc = acc + nm @ An[16 * m:16 * (m + 1), :]
    o = jnp.maximum(acc, 0.0) @ W2_r[...] + b2_r[...]
    S1 = S1_r[...]
    h = sf @ S1[0:128] + o @ S1[128:256] + Sb1_r[...]
    h = _ln(h, Sg_r[...], Sbe_r[...])
    se = jnp.maximum(h, 0.0) @ S2_r[...] + Sb2_r[...]
    t = jnp.maximum(o @ V1_r[...] + V1b_r[...], 0.0)
    vo2 = t @ V2_r[...] + V2b_r[...]
    vmx = jnp.zeros_like(vx)
    vmy = jnp.zeros_like(vy)
    vmz = jnp.zeros_like(vz)
    for s in range(16):
        blk = vo2[:, 16 * s:16 * (s + 1)]
        vmx = vmx + blk * lx[:, s:s + 1]
        vmy = vmy + blk * ly[:, s:s + 1]
        vmz = vmz + blk * lz[:, s:s + 1]
    WT = WT_r[...]
    UT = UT_r[...]
    pmx = vx @ WT[0:16] + vmx @ WT[16:32]
    pmy = vy @ WT[0:16] + vmy @ WT[16:32]
    pmz = vz @ WT[0:16] + vmz @ WT[16:32]
    ddx = vx @ UT[0:16] + vmx @ UT[16:32]
    ddy = vy @ UT[0:16] + vmy @ UT[16:32]
    ddz = vz @ UT[0:16] + vmz @ UT[16:32]
    dot = pmx * ddx + pmy * ddy + pmz * ddz
    d2 = ddx * ddx + ddy * ddy + ddz * ddz
    f = dot / (d2 + EPS)
    ok = dot >= 0.0
    vex = SLOPE * pmx + (1.0 - SLOPE) * jnp.where(ok, pmx, pmx - f * ddx)
    vey = SLOPE * pmy + (1.0 - SLOPE) * jnp.where(ok, pmy, pmy - f * ddy)
    vez = SLOPE * pmz + (1.0 - SLOPE) * jnp.where(ok, pmz, pmz - f * ddz)
    vn_ve = jnp.sqrt(vex * vex + vey * vey + vez * vez + EPS)
    Q1 = Q1_r[...]
    hq = se @ Q1[0:128] + vn_ve @ Q1[128:144] + Qb1_r[...]
    hq = _ln(hq, Qg_r[...], Qbe_r[...])
    qv = jnp.maximum(hq, 0.0) @ Q2_r[...] + Qb2_r[...]
    st_ref[:, 0:128] = se
    st_ref[:, 128:144] = vex
    st_ref[:, 144:160] = vey
    st_ref[:, 160:176] = vez
    q_ref[...] = qv


def _node_pre(sf, vfx, vfy, vfz, lsx, lsy, lsz, iv, wts):
    n = sf.shape[0]
    bn = 1000 if n % 1000 == 0 else n
    grid = (n // bn,)
    node_ins = [sf, vfx, vfy, vfz, lsx, lsy, lsz, iv]
    node_specs = [_row_spec(bn, a.shape[1]) for a in node_ins]
    w_specs = [_full_spec(w.shape) for w in wts]
    return pl.pallas_call(
        _node_pre_body,
        grid=grid,
        in_specs=node_specs + w_specs,
        out_specs=[_row_spec(bn, 256), _row_spec(bn, 128)],
        out_shape=[jax.ShapeDtypeStruct((n, 256), jnp.float32),
                   jax.ShapeDtypeStruct((n, 128), jnp.float32)],
    )(*node_ins, *wts)


def _edge_gather(src_table, qtab, src, dst):
    e = src.shape[0]
    nw = 32
    per = e // nw
    c = 200
    nch = per // c
    mesh = plsc.VectorSubcoreMesh(core_axis_name="c", subcore_axis_name="s")

    @functools.partial(
        pl.kernel,
        out_type=(jax.ShapeDtypeStruct((e, 256), jnp.float32),
                  jax.ShapeDtypeStruct((e, 128), jnp.float32)),
        mesh=mesh,
        scratch_types=[pltpu.VMEM((c,), jnp.int32), pltpu.VMEM((c,), jnp.int32),
                       pltpu.VMEM((c, 256), jnp.float32),
                       pltpu.VMEM((c, 128), jnp.float32),
                       pltpu.SemaphoreType.DMA],
    )
    def k(st_hbm, qt_hbm, src_hbm, dst_hbm, sg_hbm, qd_hbm,
          si_v, di_v, r1_v, r2_v, sem):
        wid = lax.axis_index("c") * 16 + lax.axis_index("s")
        base = wid * per

        def body(i, carry):
            off = base + i * c
            pltpu.sync_copy(src_hbm.at[pl.ds(off, c)], si_v)
            pltpu.sync_copy(dst_hbm.at[pl.ds(off, c)], di_v)
            d1 = pltpu.async_copy(st_hbm.at[si_v], r1_v, sem)
            d2 = pltpu.async_copy(qt_hbm.at[di_v], r2_v, sem)
            d1.wait()
            d2.wait()
            pltpu.sync_copy(r1_v, sg_hbm.at[pl.ds(off, c)])
            pltpu.sync_copy(r2_v, qd_hbm.at[pl.ds(off, c)])
            return carry

        lax.fori_loop(0, nch, body, 0)

    return k(src_table, qtab, src, dst)


def _edge_body(sg_r, qd_r, rf_r, ef_r, rx_r,
               Wh1a_r, Wh1b_r, Ws1a_r, Ws1b_r, Ws1c_r, Ws1d_r, bs1_r,
               Wv1_r, Wg1_r, bg1_r,
               Wh2_r, Ws2a_r, Ws2b_r, bs2_r, Wv2_r, Wg2_r, bg2_r,
               Wh3_r, Ws3a_r, Ws3b_r, bs3_r, Wv3_r, Wg3_r, bg3_r,
               K1a_r, K1b_r, Kb1_r, Kg_r, Kbe_r, K2_r, Kb2_r,
               fm_ref, fv_ref, lg_ref, lmax_ref):
    sg = sg_r[...]
    sc = sg[:, 0:128]
    vsx = sg[:, 128:144]
    vsy = sg[:, 144:160]
    vsz = sg[:, 160:176]
    rx = rx_r[...]
    Wh1b = Wh1b_r[...]
    vhx = vsx @ Wh1a_r[...] + rx[:, 0:1] * Wh1b
    vhy = vsy @ Wh1a_r[...] + rx[:, 1:2] * Wh1b
    vhz = vsz @ Wh1a_r[...] + rx[:, 2:3] * Wh1b
    vn = jnp.sqrt(vhx * vhx + vhy * vhy + vhz * vhz + EPS)
    s = jnp.maximum(sc @ Ws1a_r[...] + rf_r[...] @ Ws1b_r[...]
                    + ef_r[...] @ Ws1c_r[...] + vn @ Ws1d_r[...] + bs1_r[...], 0.0)
    gate = jax.nn.sigmoid(s @ Wg1_r[...] + bg1_r[...])
    vx = (vhx @ Wv1_r[...]) * gate
    vy = (vhy @ Wv1_r[...]) * gate
    vz = (vhz @ Wv1_r[...]) * gate
    vhx = vx @ Wh2_r[...]
    vhy = vy @ Wh2_r[...]
    vhz = vz @ Wh2_r[...]
    vn = jnp.sqrt(vhx * vhx + vhy * vhy + vhz * vhz + EPS)
    s = jnp.maximum(s @ Ws2a_r[...] + vn @ Ws2b_r[...] + bs2_r[...], 0.0)
    gate = jax.nn.sigmoid(s @ Wg2_r[...] + bg2_r[...])
    vx = (vhx @ Wv2_r[...]) * gate
    vy = (vhy @ Wv2_r[...]) * gate
    vz = (vhz @ Wv2_r[...]) * gate
    vhx = vx @ Wh3_r[...]
    vhy = vy @ Wh3_r[...]
    vhz = vz @ Wh3_r[...]
    vn = jnp.sqrt(vhx * vhx + vhy * vhy + vhz * vhz + EPS)
    s = jnp.maximum(s @ Ws3a_r[...] + vn @ Ws3b_r[...] + bs3_r[...], 0.0)
    gate = jax.nn.sigmoid(s @ Wg3_r[...] + bg3_r[...])
    vx = (vhx @ Wv3_r[...]) * gate
    vy = (vhy @ Wv3_r[...]) * gate
    vz = (vhz @ Wv3_r[...]) * gate
    vn = jnp.sqrt(vx * vx + vy * vy + vz * vz + EPS)
    h = s @ K1a_r[...] + vn @ K1b_r[...] + Kb1_r[...]
    h = _ln(h, Kg_r[...], Kbe_r[...])
    kk = jnp.maximum(h, 0.0) @ K2_r[...] + Kb2_r[...]
    lg = jnp.sum(qd_r[...] * kk, axis=1, keepdims=True) * float(1.0 / np.sqrt(128.0))
    fm_ref[...] = s
    fv_ref[:, 0:16] = vx
    fv_ref[:, 16:32] = vy
    fv_ref[:, 32:48] = vz
    one0 = (lax.broadcasted_iota(jnp.int32, (vx.shape[0], 16), 1) == 0)
    fv_ref[:, 48:64] = one0.astype(jnp.float32)
    fv_ref[:, 64:128] = jnp.zeros((vx.shape[0], 64), jnp.float32)
    lg_ref[...] = lg
    bmax = jnp.max(lg, axis=0, keepdims=True)
    lmax_ref[...] = jnp.where(pl.program_id(0) == 0, bmax,
                              jnp.maximum(lmax_ref[...], bmax))


def _edge_compute(sg, qd, rf, ef, rx, wts):
    e = sg.shape[0]
    be = 2000 if e % 2000 == 0 else e
    nblk = e // be
    edge_ins = [sg, qd, rf, ef, rx]
    edge_specs = [_row_spec(be, a.shape[1]) for a in edge_ins]
    w_specs = [_full_spec(w.shape) for w in wts]
    return pl.pallas_call(
        _edge_body,
        grid=(nblk,),
        in_specs=edge_specs + w_specs,
        out_specs=[_row_spec(be, 128), _row_spec(be, 128), _row_spec(be, 1),
                   pl.BlockSpec((1, 1), lambda i: (0, 0))],
        out_shape=[jax.ShapeDtypeStruct((e, 128), jnp.float32),
                   jax.ShapeDtypeStruct((e, 128), jnp.float32),
                   jax.ShapeDtypeStruct((e, 1), jnp.float32),
                   jax.ShapeDtypeStruct((1, 1), jnp.float32)],
    )(*edge_ins, *wts)


def _edge_scale_body(fm_r, fv_r, lg_r, gm_r, fms_ref, fvs_ref):
    ex = jnp.exp(lg_r[...] - gm_r[...])
    fms_ref[...] = fm_r[...] * ex
    fvs_ref[...] = fv_r[...] * ex


def _edge_scale(fm, fv, lg, gmax11):
    e = fm.shape[0]
    be = 8000 if e % 8000 == 0 else e
    return pl.pallas_call(
        _edge_scale_body,
        grid=(e // be,),
        in_specs=[_row_spec(be, 128), _row_spec(be, 128), _row_spec(be, 1),
                  pl.BlockSpec((1, 1), lambda i: (0, 0))],
        out_specs=[_row_spec(be, 128), _row_spec(be, 128)],
        out_shape=[jax.ShapeDtypeStruct((e, 128), jnp.float32),
                   jax.ShapeDtypeStruct((e, 128), jnp.float32)],
    )(fm, fv, lg, gmax11)


def _edge_aggregate(feat, dst, zrows):
    e, w = feat.shape
    npad = zrows.shape[0]
    nw = 32
    per = e // nw
    c = 200
    nch = per // c
    rpt = npad // 16
    mesh = plsc.VectorSubcoreMesh(core_axis_name="c", subcore_axis_name="s")

    @functools.partial(
        pl.kernel,
        out_type=jax.ShapeDtypeStruct((2, npad, w), jnp.float32),
        mesh=mesh,
        scratch_types=[pltpu.VMEM((c,), jnp.int32),
                       pltpu.VMEM((c, w), jnp.float32),
                       pltpu.VMEM_SHARED((npad, w), jnp.float32),
                       pltpu.SemaphoreType.DMA],
    )
    def k(f_hbm, dst_hbm, z_hbm, out_hbm, idxb, fb, sh, sem):
        cid = lax.axis_index("c")
        sid = lax.axis_index("s")
        base = (cid * 16 + sid) * per

        pltpu.sync_copy(z_hbm.at[pl.ds(sid * rpt, rpt)],
                        sh.at[pl.ds(sid * rpt, rpt)])
        plsc.subcore_barrier()

        def chunk(i, carry):
            off = base + i * c
            pltpu.sync_copy(dst_hbm.at[pl.ds(off, c)], idxb)
            pltpu.sync_copy(f_hbm.at[pl.ds(off, c)], fb)
            pltpu.sync_copy(fb, sh.at[idxb], add=True)
            return carry

        lax.fori_loop(0, nch, chunk, 0)
        plsc.subcore_barrier()
        pltpu.sync_copy(sh.at[pl.ds(sid * rpt, rpt)],
                        out_hbm.at[cid, pl.ds(sid * rpt, rpt)])

    return k(feat, dst, zrows)


def _node_post_body(p00_r, p10_r, p01_r, p11_r, st_r, lig_r, sf_r, vfx_r, vfy_r, vfz_r, xx_r,
                    NA_r, NB_r, NC_r, W1_r, W2_r, W3_r, W4_r, nb1_r,
                    Nv1_r, Ng1_r, ngb1_r,
                    Mh2_r, M2a_r, M2b_r, mb2_r, Mv2_r, Mg2_r, mgb2_r,
                    Mh3_r, M3a_r, M3b_r, mb3_r, Mv3_r, Mg3_r, mgb3_r,
                    G_r, B_r,
                    s_ref, vxo_ref, vyo_ref, vzo_ref):
    aggs = p00_r[...] + p10_r[...]
    aggv = p01_r[...] + p11_r[...]
    inv = 1.0 / (aggv[:, 48:49] + EPS)
    sagg = aggs * inv
    vax = aggv[:, 0:16] * inv
    vay = aggv[:, 16:32] * inv
    vaz = aggv[:, 32:48] * inv
    st = st_r[...]
    se = st[:, 0:128]
    vex = st[:, 128:144]
    vey = st[:, 144:160]
    vez = st[:, 160:176]
    xx = xx_r[...]
    NA = NA_r[...]
    vhx = xx[:, 0:1] * NA + vex @ NB_r[...] + vax @ NC_r[...]
    vhy = xx[:, 1:2] * NA + vey @ NB_r[...] + vay @ NC_r[...]
    vhz = xx[:, 2:3] * NA + vez @ NB_r[...] + vaz @ NC_r[...]
    vn = jnp.sqrt(vhx * vhx + vhy * vhy + vhz * vhz + EPS)
    s = jnp.maximum(lig_r[...] @ W1_r[...] + se @ W2_r[...]
                    + sagg @ W3_r[...] + vn @ W4_r[...] + nb1_r[...], 0.0)
    gate = jax.nn.sigmoid(s @ Ng1_r[...] + ngb1_r[...])
    vx = (vhx @ Nv1_r[...]) * gate
    vy = (vhy @ Nv1_r[...]) * gate
    vz = (vhz @ Nv1_r[...]) * gate
    vhx = vx @ Mh2_r[...]
    vhy = vy @ Mh2_r[...]
    vhz = vz @ Mh2_r[...]
    vn = jnp.sqrt(vhx * vhx + vhy * vhy + vhz * vhz + EPS)
    s = jnp.maximum(s @ M2a_r[...] + vn @ M2b_r[...] + mb2_r[...], 0.0)
    gate = jax.nn.sigmoid(s @ Mg2_r[...] + mgb2_r[...])
    vx = (vhx @ Mv2_r[...]) * gate
    vy = (vhy @ Mv2_r[...]) * gate
    vz = (vhz @ Mv2_r[...]) * gate
    vhx = vx @ Mh3_r[...]
    vhy = vy @ Mh3_r[...]
    vhz = vz @ Mh3_r[...]
    vn = jnp.sqrt(vhx * vhx + vhy * vhy + vhz * vhz + EPS)
    s = jnp.maximum(s @ M3a_r[...] + vn @ M3b_r[...] + mb3_r[...], 0.0)
    gate = jax.nn.sigmoid(s @ Mg3_r[...] + mgb3_r[...])
    vx = (vhx @ Mv3_r[...]) * gate
    vy = (vhy @ Mv3_r[...]) * gate
    vz = (vhz @ Mv3_r[...]) * gate
    s = sf_r[...] + s
    s = _ln(s, G_r[...], B_r[...])
    vx = vfx_r[...] + vx
    vy = vfy_r[...] + vy
    vz = vfz_r[...] + vz
    vn2 = jnp.sum(vx * vx + vy * vy + vz * vz, axis=1, keepdims=True) * (1.0 / 16.0)
    r = 1.0 / jnp.sqrt(vn2 + EPS)
    s_ref[...] = s
    vxo_ref[...] = vx * r
    vyo_ref[...] = vy * r
    vzo_ref[...] = vz * r


def _node_post(p00, p10, p01, p11, st, lig, sf, vfx, vfy, vfz, xx, wts):
    n = sf.shape[0]
    bn = 1000 if n % 1000 == 0 else n
    node_ins = [p00, p10, p01, p11, st, lig, sf, vfx, vfy, vfz, xx]
    node_specs = [_row_spec(bn, a.shape[1]) for a in node_ins]
    w_specs = [_full_spec(w.shape) for w in wts]
    return pl.pallas_call(
        _node_post_body,
        grid=(n // bn,),
        in_specs=node_specs + w_specs,
        out_specs=[_row_spec(bn, 128), _row_spec(bn, 16),
                   _row_spec(bn, 16), _row_spec(bn, 16)],
        out_shape=[jax.ShapeDtypeStruct((n, 128), jnp.float32),
                   jax.ShapeDtypeStruct((n, 16), jnp.float32),
                   jax.ShapeDtypeStruct((n, 16), jnp.float32),
                   jax.ShapeDtypeStruct((n, 16), jnp.float32)],
    )(*node_ins, *wts)


def _prep_pre_weights(p):
    se1w = p['se1']['w']
    wts = [se1w[0:128], se1w[128:384], se1w[384:400], se1w[400:416],
           p['se1']['b'][None], p['se2']['w'], p['se2']['b'][None]]
    ss = p['shape_scalar']
    wts += [ss['l1']['w'], ss['l1']['b'][None], ss['g'][None], ss['be'][None],
            ss['l2']['w'], ss['l2']['b'][None]]
    perm = np.arange(256).reshape(16, 16).T.reshape(-1)
    wts += [p['sv1']['w'], p['sv1']['b'][None],
            p['sv2']['w'][:, perm], p['sv2']['b'][perm][None]]
    wts += [p['vec_emb']['W'].T, p['vec_emb']['U'].T]
    hq = p['hq']
    wts += [hq['l1']['w'], hq['l1']['b'][None], hq['g'][None], hq['be'][None],
            hq['l2']['w'], hq['l2']['b'][None]]
    return wts


def _prep_edge_weights(p):
    g1, g2, g3 = p['mess']
    wh1t = g1['wh'].T
    ws1 = g1['ws']['w']
    wts = [wh1t[0:16], wh1t[16:17],
           ws1[0:128], ws1[128:208], ws1[208:212], ws1[212:228],
           g1['ws']['b'][None], g1['wv'].T, g1['wg']['w'], g1['wg']['b'][None]]
    for g in (g2, g3):
        ws = g['ws']['w']
        wts += [g['wh'].T, ws[0:128], ws[128:144], g['ws']['b'][None],
                g['wv'].T, g['wg']['w'], g['wg']['b'][None]]
    hk = p['hk']
    k1 = hk['l1']['w']
    wts += [k1[0:128], k1[128:144], hk['l1']['b'][None], hk['g'][None],
            hk['be'][None], hk['l2']['w'], hk['l2']['b'][None]]
    return wts


def _prep_node_weights(p):
    n1, n2, n3 = p['node']
    wh1t = n1['wh'].T
    ws1 = n1['ws']['w']
    wts = [wh1t[0:1], wh1t[1:17], wh1t[17:33],
           ws1[0:128], ws1[128:256], ws1[256:384], ws1[384:512],
           n1['ws']['b'][None], n1['wv'].T, n1['wg']['w'], n1['wg']['b'][None]]
    for g in (n2, n3):
        ws = g['ws']['w']
        wts += [g['wh'].T, ws[0:128], ws[128:256], g['ws']['b'][None],
                g['wv'].T, g['wg']['w'], g['wg']['b'][None]]
    wts += [p['nn_g'][None], p['nn_b'][None]]
    return wts


def kernel(scalar_feat, vec_feat, r_feat, rel_x, x, ligand_emb, edge_feat,
           edge_index, ligand_shape, invar_ligand_shape, params):
    n = scalar_feat.shape[0]
    e = r_feat.shape[0]
    vft = jnp.transpose(vec_feat, (2, 0, 1))
    lst = jnp.transpose(ligand_shape, (2, 0, 1))
    src = edge_index[0].astype(jnp.int32)
    dst = edge_index[1].astype(jnp.int32)

    st, q = _node_pre(scalar_feat, vft[0], vft[1], vft[2],
                      lst[0], lst[1], lst[2], invar_ligand_shape,
                      _prep_pre_weights(params))
    sg, qd = _edge_gather(st, q, src, dst)
    fm, fv, lg, lmax = _edge_compute(sg, qd, r_feat, edge_feat, rel_x,
                                     _prep_edge_weights(params))
    fms, fvs = _edge_scale(fm, fv, lg, lmax)
    npad = ((n + 127) // 128) * 128
    zrows = jnp.zeros((npad, 128), jnp.float32)
    ps = _edge_aggregate(fms, dst, zrows)
    pv = _edge_aggregate(fvs, dst, zrows)

    s, vx, vy, vz = _node_post(ps[0, :n], ps[1, :n],
                               pv[0, :n], pv[1, :n],
                               st, ligand_emb,
                               scalar_feat, vft[0], vft[1], vft[2],
                               x.reshape(n, 3), _prep_node_weights(params))
    v = jnp.stack([vx, vy, vz], axis=-1)
    return (s, v)
